# baseline (speedup 1.0000x reference)
import functools
import jax
import jax.numpy as jnp
from jax import lax
from jax.experimental import pallas as pl
from jax.experimental.pallas import tpu as pltpu
from jax.experimental.pallas import tpu_sc as plsc

BATCH = 16384
VOCAB = 100000
DIM = 64
LANES = 16

NUM_CORES = 2
NUM_SUBCORES = 16
NW = NUM_CORES * NUM_SUBCORES
B_PER_W = BATCH // NW
GCHUNK = 128
NCHUNK = B_PER_W // GCHUNK


def _body(x_hbm, idx_hbm, emb_hbm, out_hbm, idx_v, rows_v, x_v, gsem):
    wid = lax.axis_index("s") * NUM_CORES + lax.axis_index("c")
    base = wid * B_PER_W

    pltpu.sync_copy(idx_hbm.at[pl.ds(base, B_PER_W)], idx_v)

    descs = []
    for j in range(NCHUNK):
        descs.append(
            pltpu.async_copy(
                emb_hbm.at[idx_v.at[pl.ds(j * GCHUNK, GCHUNK)]],
                rows_v.at[pl.ds(j * GCHUNK, GCHUNK)],
                gsem,
            )
        )
    pltpu.sync_copy(x_hbm.at[pl.ds(base, B_PER_W)], x_v)
    for d in descs:
        d.wait()

    def add_row(i, carry):
        for j in range(DIM // LANES):
            sl = pl.ds(j * LANES, LANES)
            plsc.addupdate(rows_v.at[i, sl], x_v[i, sl])
        return carry

    lax.fori_loop(0, B_PER_W, add_row, 0)

    pltpu.sync_copy(rows_v, out_hbm.at[pl.ds(base, B_PER_W)])


@jax.jit
def _ffoverlay(X, y_true, embedding):
    mesh = plsc.VectorSubcoreMesh(core_axis_name="c", subcore_axis_name="s")
    run = pl.kernel(
        _body,
        out_type=jax.ShapeDtypeStruct((BATCH, DIM), jnp.float32),
        mesh=mesh,
        scratch_types=[
            pltpu.VMEM((B_PER_W,), jnp.int32),
            pltpu.VMEM((B_PER_W, DIM), jnp.float32),
            pltpu.VMEM((B_PER_W, DIM), jnp.float32),
            pltpu.SemaphoreType.DMA,
        ],
        compiler_params=pltpu.CompilerParams(use_tc_tiling_on_sc=False),
    )
    return run(X, y_true, embedding)


def kernel(X, y_true, embedding):
    return _ffoverlay(X, y_true.astype(jnp.int32), embedding)


# speedup vs baseline: 1.0179x; 1.0179x over previous
import functools
import jax
import jax.numpy as jnp
from jax import lax
from jax.experimental import pallas as pl
from jax.experimental.pallas import tpu as pltpu
from jax.experimental.pallas import tpu_sc as plsc

BATCH = 16384
VOCAB = 100000
DIM = 64
LANES = 16

NUM_CORES = 2
NUM_SUBCORES = 16
NW = NUM_CORES * NUM_SUBCORES
B_PER_W = BATCH // NW
GCHUNK = 128
NCHUNK = B_PER_W // GCHUNK


def _body(x_hbm, idx_hbm, emb_hbm, out_hbm, idx_v, rows_v, gsem):
    wid = lax.axis_index("s") * NUM_CORES + lax.axis_index("c")
    base = wid * B_PER_W

    pltpu.sync_copy(idx_hbm.at[pl.ds(base, B_PER_W)], idx_v)
    pltpu.sync_copy(x_hbm.at[pl.ds(base, B_PER_W)], rows_v)

    descs = []
    for j in range(NCHUNK):
        descs.append(
            pltpu.async_copy(
                emb_hbm.at[idx_v.at[pl.ds(j * GCHUNK, GCHUNK)]],
                rows_v.at[pl.ds(j * GCHUNK, GCHUNK)],
                gsem,
                add=True,
            )
        )
    for d in descs:
        d.wait()

    pltpu.sync_copy(rows_v, out_hbm.at[pl.ds(base, B_PER_W)])


@jax.jit
def _ffoverlay(X, y_true, embedding):
    mesh = plsc.VectorSubcoreMesh(core_axis_name="c", subcore_axis_name="s")
    run = pl.kernel(
        _body,
        out_type=jax.ShapeDtypeStruct((BATCH, DIM), jnp.float32),
        mesh=mesh,
        scratch_types=[
            pltpu.VMEM((B_PER_W,), jnp.int32),
            pltpu.VMEM((B_PER_W, DIM), jnp.float32),
            pltpu.SemaphoreType.DMA,
        ],
        compiler_params=pltpu.CompilerParams(use_tc_tiling_on_sc=False),
    )
    return run(X, y_true, embedding)


def kernel(X, y_true, embedding):
    return _ffoverlay(X, y_true.astype(jnp.int32), embedding)
